# baseline (speedup 1.0000x reference)
import jax
import jax.numpy as jnp
from jax.experimental import pallas as pl
from jax.experimental.pallas import tpu as pltpu
from jax.experimental.pallas import tpu_sc as plsc

_BF = jnp.bfloat16
_F32 = jnp.float32


_NC, _NS = 2, 16
_NW = _NC * _NS


def _sc_gather_all(bat2, b_idx, pit2, p_idx, team_pad, t_idx):
    nb, np_, nt = b_idx.shape[0], p_idx.shape[0], t_idx.shape[0]
    bpw, ppw, tpw = nb // _NW, np_ // _NW, nt // _NW
    mesh = plsc.VectorSubcoreMesh(core_axis_name="c", subcore_axis_name="s")

    out_types = (
        jax.ShapeDtypeStruct((nb, 128), bat2.dtype),
        jax.ShapeDtypeStruct((np_, 128), pit2.dtype),
        jax.ShapeDtypeStruct((nt, 128), team_pad.dtype),
    )
    scratch = [
        pltpu.VMEM((bpw,), jnp.int32),
        pltpu.VMEM((bpw, 128), jnp.float32),
        pltpu.VMEM((ppw,), jnp.int32),
        pltpu.VMEM((ppw, 128), jnp.float32),
        pltpu.VMEM((tpw,), jnp.int32),
        pltpu.VMEM((tpw, 128), jnp.float32),
        pltpu.SemaphoreType.DMA,
    ]

    @pl.kernel(out_type=out_types, mesh=mesh, scratch_types=scratch)
    def k(bat_hbm, bidx_hbm, pit_hbm, pidx_hbm, team_hbm, tidx_hbm,
          ob_hbm, op_hbm, ot_hbm,
          bidx_v, brows_v, pidx_v, prows_v, tidx_v, trows_v, sem):
        wid = jax.lax.axis_index("s") * _NC + jax.lax.axis_index("c")

        tbase = wid * tpw
        pltpu.sync_copy(tidx_hbm.at[pl.ds(tbase, tpw)], tidx_v)
        pltpu.async_copy(team_hbm.at[tidx_v], trows_v, sem).wait()
        pltpu.sync_copy(trows_v, ot_hbm.at[pl.ds(tbase, tpw)])

        pbase = wid * ppw
        pltpu.sync_copy(pidx_hbm.at[pl.ds(pbase, ppw)], pidx_v)
        pltpu.async_copy(pit_hbm.at[pidx_v], prows_v, sem).wait()
        pltpu.sync_copy(prows_v, op_hbm.at[pl.ds(pbase, ppw)])

        bbase = wid * bpw
        pltpu.sync_copy(bidx_hbm.at[pl.ds(bbase, bpw)], bidx_v)
        pltpu.async_copy(bat_hbm.at[bidx_v], brows_v, sem).wait()
        pltpu.sync_copy(brows_v, ob_hbm.at[pl.ds(bbase, bpw)])

    return k(bat2, b_idx, pit2, p_idx, team_pad, t_idx)


def _l1_d1(xp, xq, Wp, bp, Wq, bq, W1, b1):
    B, Kp = xp.shape
    Kq = xq.shape[1]
    H = Wp.shape[1]
    N = W1.shape[1]
    KT = 512
    nk = H // KT

    def kern(xp_ref, xq_ref, wp_ref, bp_ref, wq_ref, bq_ref, w1_ref, b1_ref,
             o_ref, acc_ref):
        k = pl.program_id(0)

        @pl.when(k == 0)
        def _():
            acc_ref[...] = jnp.zeros_like(acc_ref)

        w1t = w1_ref[...].astype(_BF)
        hp = jnp.dot(xp_ref[...], wp_ref[...].astype(_BF),
                     preferred_element_type=_F32)
        hp = jnp.maximum(hp + bp_ref[...], 0.0).astype(_BF)
        acc_ref[0:B, :] += jnp.dot(hp, w1t, preferred_element_type=_F32)
        hq = jnp.dot(xq_ref[...], wq_ref[...].astype(_BF),
                     preferred_element_type=_F32)
        hq = jnp.maximum(hq + bq_ref[...], 0.0).astype(_BF)
        acc_ref[B:2 * B, :] += jnp.dot(hq, w1t, preferred_element_type=_F32)

        @pl.when(k == nk - 1)
        def _():
            o_ref[...] = jnp.maximum(acc_ref[...] + b1_ref[...], 0.0).astype(_BF)

    return pl.pallas_call(
        kern,
        grid=(nk,),
        in_specs=[
            pl.BlockSpec((B, Kp), lambda k: (0, 0)),
            pl.BlockSpec((B, Kq), lambda k: (0, 0)),
            pl.BlockSpec((Kp, KT), lambda k: (0, k)),
            pl.BlockSpec((1, KT), lambda k: (0, k)),
            pl.BlockSpec((Kq, KT), lambda k: (0, k)),
            pl.BlockSpec((1, KT), lambda k: (0, k)),
            pl.BlockSpec((KT, N), lambda k: (k, 0)),
            pl.BlockSpec((1, N), lambda k: (0, 0)),
        ],
        out_specs=pl.BlockSpec((2 * B, N), lambda k: (0, 0)),
        out_shape=jax.ShapeDtypeStruct((2 * B, N), _BF),
        scratch_shapes=[pltpu.VMEM((2 * B, N), _F32)],
    )(xp, xq, Wp, bp.reshape(1, -1), Wq, bq.reshape(1, -1), W1,
      b1.reshape(1, -1))


def _d2_d3_heads(x, W2, b2, W3, b3, Wh, bh):
    M, K = x.shape
    MT = 512

    def kern(x_ref, w2_ref, b2_ref, w3_ref, b3_ref, wh_ref, bh_ref, o_ref):
        t = jnp.dot(x_ref[...], w2_ref[...].astype(_BF),
                    preferred_element_type=_F32)
        t = jnp.maximum(t + b2_ref[...], 0.0).astype(_BF)
        h = jnp.dot(t, w3_ref[...].astype(_BF), preferred_element_type=_F32)
        h = jnp.maximum(h + b3_ref[...], 0.0)
        o = jnp.dot(h.astype(_BF), wh_ref[...].astype(_BF),
                    preferred_element_type=_F32)
        o = o + bh_ref[...]
        col = jax.lax.broadcasted_iota(jnp.int32, o.shape, 1)
        o = o + jnp.where((col == 11) | (col == 16) | (col == 17), -999.0, 0.0)
        o = jnp.where(col == 20, jax.nn.sigmoid(o), o)
        o_ref[...] = o

    return pl.pallas_call(
        kern,
        grid=(M // MT,),
        in_specs=[
            pl.BlockSpec((MT, K), lambda m: (m, 0)),
            pl.BlockSpec(W2.shape, lambda m: (0, 0)),
            pl.BlockSpec((1, b2.shape[0]), lambda m: (0, 0)),
            pl.BlockSpec(W3.shape, lambda m: (0, 0)),
            pl.BlockSpec((1, b3.shape[0]), lambda m: (0, 0)),
            pl.BlockSpec(Wh.shape, lambda m: (0, 0)),
            pl.BlockSpec((1, Wh.shape[1]), lambda m: (0, 0)),
        ],
        out_specs=pl.BlockSpec((MT, Wh.shape[1]), lambda m: (m, 0)),
        out_shape=jax.ShapeDtypeStruct((M, Wh.shape[1]), _F32),
    )(x, W2, b2.reshape(1, -1), W3, b3.reshape(1, -1), Wh, bh)


def kernel(outs_ct, bat_id, pit_id, fld_team_id, base1_run_id, base2_run_id,
           base3_run_id, away_score_ct, home_score_ct, inn_ct, bat_home_id,
           away_bat_lineup, home_bat_lineup, away_start_bat_ids,
           home_start_bat_ids, away_pit_id, home_pit_id, away_team_id,
           home_team_id, bat_table, pit_table, team_table, policy_W, policy_b,
           pred_W, pred_b, d1_W, d1_b, d2_W, d2_b, d3_W, d3_b, batd_W, batd_b,
           r1_W, r1_b, r2_W, r2_b, r3_W, r3_b, po_W, po_b):
    B = bat_id.shape[0]
    i32 = jnp.int32

    b_idx = jnp.concatenate(
        [bat_id, base1_run_id, base2_run_id, base3_run_id,
         away_start_bat_ids.T.reshape(-1),
         home_start_bat_ids.T.reshape(-1)]).astype(i32)
    p_idx = jnp.concatenate([pit_id, away_pit_id, home_pit_id]).astype(i32)
    team_idx = jnp.concatenate(
        [fld_team_id, away_team_id, home_team_id]).astype(i32)

    gb2, gp2, gt = _sc_gather_all(
        bat_table.reshape(-1, 128), b_idx >> 1,
        pit_table.reshape(-1, 128), p_idx >> 1,
        jnp.pad(team_table, ((0, 0), (0, 64))), team_idx)

    gb = jnp.where((b_idx & 1)[:, None] == 1, gb2[:, 64:], gb2[:, :64])
    gp = jnp.where((p_idx & 1)[:, None] == 1, gp2[:, 64:], gp2[:, :64])

    def bslot(s):
        return gb[s * B:(s + 1) * B].astype(_BF)

    def pslot(s):
        return gp[s * B:(s + 1) * B].astype(_BF)

    def tslot(s):
        return gt[s * B:(s + 1) * B, :64].astype(_BF)

    x_pred = jnp.concatenate(
        [outs_ct.astype(_BF), bslot(0), pslot(0), tslot(0),
         bslot(1), bslot(2), bslot(3),
         away_score_ct.astype(_BF), home_score_ct.astype(_BF),
         inn_ct.astype(_BF), bat_home_id.astype(_BF),
         away_bat_lineup.astype(_BF), home_bat_lineup.astype(_BF)]
        + [bslot(4 + k) for k in range(9)]
        + [bslot(13 + k) for k in range(9)]
        + [pslot(1), pslot(2), tslot(1), tslot(2)], axis=1)
    x_policy = x_pred[:, :385]

    h2 = _l1_d1(x_policy, x_pred, policy_W, policy_b, pred_W, pred_b,
                d1_W, d1_b)

    Wh = jnp.concatenate([batd_W, r1_W, r2_W, r3_W, po_W], axis=1)
    Wh = jnp.pad(Wh, ((0, 0), (0, 128 - Wh.shape[1])))
    bh = jnp.concatenate([batd_b, r1_b, r2_b, r3_b, po_b])
    bh = jnp.pad(bh, (0, 128 - bh.shape[0])).reshape(1, 128)

    out = _d2_d3_heads(h2, d2_W, d2_b, d3_W, d3_b, Wh, bh)

    bat_dest = out[:B, 0:5]
    run1_dest = out[:B, 5:10]
    run2_dest = out[:B, 10:15]
    run3_dest = out[:B, 15:20]
    pred = out[B:, 20:21]
    return (bat_dest, run1_dest, run2_dest, run3_dest, pred)


# speedup vs baseline: 1.0990x; 1.0990x over previous
import jax
import jax.numpy as jnp
from jax.experimental import pallas as pl
from jax.experimental.pallas import tpu as pltpu
from jax.experimental.pallas import tpu_sc as plsc

_BF = jnp.bfloat16
_F32 = jnp.float32


_NC, _NS = 2, 16
_NW = _NC * _NS


def _sc_gather_all(comb, bp_idx, team_pad, team_idx):
    emb = comb.shape[1]
    nb = bp_idx.shape[0]
    nt = team_idx.shape[0]
    bpw = nb // _NW
    tpw = nt // _NW
    mesh = plsc.VectorSubcoreMesh(core_axis_name="c", subcore_axis_name="s")

    out_types = (
        jax.ShapeDtypeStruct((nb, emb), comb.dtype),
        jax.ShapeDtypeStruct((nt, emb), team_pad.dtype),
    )
    scratch = [
        pltpu.VMEM((bpw,), jnp.int32),
        pltpu.VMEM((bpw, emb), jnp.float32),
        pltpu.VMEM((tpw,), jnp.int32),
        pltpu.VMEM((tpw, emb), jnp.float32),
        pltpu.SemaphoreType.DMA,
    ]

    @pl.kernel(out_type=out_types, mesh=mesh, scratch_types=scratch)
    def k(comb_hbm, bidx_hbm, team_hbm, tidx_hbm, ob_hbm, ot_hbm,
          bidx_v, brows_v, tidx_v, trows_v, sem):
        wid = jax.lax.axis_index("s") * _NC + jax.lax.axis_index("c")

        tbase = wid * tpw
        pltpu.sync_copy(tidx_hbm.at[pl.ds(tbase, tpw)], tidx_v)
        pltpu.async_copy(team_hbm.at[tidx_v], trows_v, sem).wait()
        pltpu.sync_copy(trows_v, ot_hbm.at[pl.ds(tbase, tpw)])

        bbase = wid * bpw
        pltpu.sync_copy(bidx_hbm.at[pl.ds(bbase, bpw)], bidx_v)
        pltpu.async_copy(comb_hbm.at[bidx_v], brows_v, sem).wait()
        pltpu.sync_copy(brows_v, ob_hbm.at[pl.ds(bbase, bpw)])

    return k(comb, bp_idx, team_pad, team_idx)


def _l1_d1(xp, xq, Wp, bp, Wq, bq, W1, b1):
    B, Kp = xp.shape
    Kq = xq.shape[1]
    H = Wp.shape[1]
    N = W1.shape[1]
    KT = 512
    nk = H // KT

    def kern(xp_ref, xq_ref, wp_ref, bp_ref, wq_ref, bq_ref, w1_ref, b1_ref,
             o_ref, acc_ref):
        k = pl.program_id(0)

        @pl.when(k == 0)
        def _():
            acc_ref[...] = jnp.zeros_like(acc_ref)

        w1t = w1_ref[...].astype(_BF)
        hp = jnp.dot(xp_ref[...], wp_ref[...].astype(_BF),
                     preferred_element_type=_F32)
        hp = jnp.maximum(hp + bp_ref[...], 0.0).astype(_BF)
        acc_ref[0:B, :] += jnp.dot(hp, w1t, preferred_element_type=_F32)
        hq = jnp.dot(xq_ref[...], wq_ref[...].astype(_BF),
                     preferred_element_type=_F32)
        hq = jnp.maximum(hq + bq_ref[...], 0.0).astype(_BF)
        acc_ref[B:2 * B, :] += jnp.dot(hq, w1t, preferred_element_type=_F32)

        @pl.when(k == nk - 1)
        def _():
            o_ref[...] = jnp.maximum(acc_ref[...] + b1_ref[...], 0.0).astype(_BF)

    return pl.pallas_call(
        kern,
        grid=(nk,),
        in_specs=[
            pl.BlockSpec((B, Kp), lambda k: (0, 0)),
            pl.BlockSpec((B, Kq), lambda k: (0, 0)),
            pl.BlockSpec((Kp, KT), lambda k: (0, k)),
            pl.BlockSpec((1, KT), lambda k: (0, k)),
            pl.BlockSpec((Kq, KT), lambda k: (0, k)),
            pl.BlockSpec((1, KT), lambda k: (0, k)),
            pl.BlockSpec((KT, N), lambda k: (k, 0)),
            pl.BlockSpec((1, N), lambda k: (0, 0)),
        ],
        out_specs=pl.BlockSpec((2 * B, N), lambda k: (0, 0)),
        out_shape=jax.ShapeDtypeStruct((2 * B, N), _BF),
        scratch_shapes=[pltpu.VMEM((2 * B, N), _F32)],
    )(xp, xq, Wp, bp.reshape(1, -1), Wq, bq.reshape(1, -1), W1,
      b1.reshape(1, -1))


def _d2_d3_heads(x, W2, b2, W3, b3, Wh, bh):
    M, K = x.shape
    MT = 512

    def kern(x_ref, w2_ref, b2_ref, w3_ref, b3_ref, wh_ref, bh_ref, o_ref):
        t = jnp.dot(x_ref[...], w2_ref[...].astype(_BF),
                    preferred_element_type=_F32)
        t = jnp.maximum(t + b2_ref[...], 0.0).astype(_BF)
        h = jnp.dot(t, w3_ref[...].astype(_BF), preferred_element_type=_F32)
        h = jnp.maximum(h + b3_ref[...], 0.0)
        o = jnp.dot(h.astype(_BF), wh_ref[...].astype(_BF),
                    preferred_element_type=_F32)
        o = o + bh_ref[...]
        col = jax.lax.broadcasted_iota(jnp.int32, o.shape, 1)
        o = o + jnp.where((col == 11) | (col == 16) | (col == 17), -999.0, 0.0)
        o = jnp.where(col == 20, jax.nn.sigmoid(o), o)
        o_ref[...] = o

    return pl.pallas_call(
        kern,
        grid=(M // MT,),
        in_specs=[
            pl.BlockSpec((MT, K), lambda m: (m, 0)),
            pl.BlockSpec(W2.shape, lambda m: (0, 0)),
            pl.BlockSpec((1, b2.shape[0]), lambda m: (0, 0)),
            pl.BlockSpec(W3.shape, lambda m: (0, 0)),
            pl.BlockSpec((1, b3.shape[0]), lambda m: (0, 0)),
            pl.BlockSpec(Wh.shape, lambda m: (0, 0)),
            pl.BlockSpec((1, Wh.shape[1]), lambda m: (0, 0)),
        ],
        out_specs=pl.BlockSpec((MT, Wh.shape[1]), lambda m: (m, 0)),
        out_shape=jax.ShapeDtypeStruct((M, Wh.shape[1]), _F32),
    )(x, W2, b2.reshape(1, -1), W3, b3.reshape(1, -1), Wh, bh)


def kernel(outs_ct, bat_id, pit_id, fld_team_id, base1_run_id, base2_run_id,
           base3_run_id, away_score_ct, home_score_ct, inn_ct, bat_home_id,
           away_bat_lineup, home_bat_lineup, away_start_bat_ids,
           home_start_bat_ids, away_pit_id, home_pit_id, away_team_id,
           home_team_id, bat_table, pit_table, team_table, policy_W, policy_b,
           pred_W, pred_b, d1_W, d1_b, d2_W, d2_b, d3_W, d3_b, batd_W, batd_b,
           r1_W, r1_b, r2_W, r2_b, r3_W, r3_b, po_W, po_b):
    B = bat_id.shape[0]
    i32 = jnp.int32

    bp_idx = jnp.concatenate(
        [bat_id, pit_id, base1_run_id, base2_run_id, base3_run_id,
         away_start_bat_ids.T.reshape(-1), home_start_bat_ids.T.reshape(-1),
         away_pit_id, home_pit_id]).astype(i32)
    team_idx = jnp.concatenate(
        [fld_team_id, away_team_id, home_team_id]).astype(i32)

    comb = jnp.concatenate([bat_table, pit_table], axis=1)
    g, gt = _sc_gather_all(comb, bp_idx,
                           jnp.pad(team_table, ((0, 0), (0, 64))), team_idx)

    def bslot(s):
        return g[s * B:(s + 1) * B, :64].astype(_BF)

    def pslot(s):
        return g[s * B:(s + 1) * B, 64:].astype(_BF)

    def tslot(s):
        return gt[s * B:(s + 1) * B, :64].astype(_BF)

    x_pred = jnp.concatenate(
        [outs_ct.astype(_BF), bslot(0), pslot(1), tslot(0),
         bslot(2), bslot(3), bslot(4),
         away_score_ct.astype(_BF), home_score_ct.astype(_BF),
         inn_ct.astype(_BF), bat_home_id.astype(_BF),
         away_bat_lineup.astype(_BF), home_bat_lineup.astype(_BF)]
        + [bslot(5 + k) for k in range(9)]
        + [bslot(14 + k) for k in range(9)]
        + [pslot(23), pslot(24), tslot(1), tslot(2)], axis=1)
    x_policy = x_pred[:, :385]

    h2 = _l1_d1(x_policy, x_pred, policy_W, policy_b, pred_W, pred_b,
                d1_W, d1_b)

    Wh = jnp.concatenate([batd_W, r1_W, r2_W, r3_W, po_W], axis=1)
    Wh = jnp.pad(Wh, ((0, 0), (0, 128 - Wh.shape[1])))
    bh = jnp.concatenate([batd_b, r1_b, r2_b, r3_b, po_b])
    bh = jnp.pad(bh, (0, 128 - bh.shape[0])).reshape(1, 128)

    out = _d2_d3_heads(h2, d2_W, d2_b, d3_W, d3_b, Wh, bh)

    bat_dest = out[:B, 0:5]
    run1_dest = out[:B, 5:10]
    run2_dest = out[:B, 10:15]
    run3_dest = out[:B, 15:20]
    pred = out[B:, 20:21]
    return (bat_dest, run1_dest, run2_dest, run3_dest, pred)
